# baseline (device time: 1413708 ns/iter reference)
import jax
import jax.numpy as jnp
from jax import lax
from jax.experimental import pallas as pl
from jax.experimental.pallas import tpu as pltpu

N_DEV = 16


def kernel(x, w_mat):
    m, k_per = x.shape
    _, n = w_mat.shape
    m_per = m // N_DEV

    def body(x_ref, w_ref, out_ref, comm_ref, send_sems, recv_sems):
        my = lax.axis_index("i")
        right = lax.rem(my + 1, N_DEV)
        left = lax.rem(my + N_DEV - 1, N_DEV)

        barrier_sem = pltpu.get_barrier_semaphore()
        for nbr in (left, right):
            pl.semaphore_signal(
                barrier_sem, inc=1,
                device_id=(nbr,), device_id_type=pl.DeviceIdType.MESH,
            )
        pl.semaphore_wait(barrier_sem, 2)

        def partial_chunk(c):
            xs = x_ref[pl.ds(c * m_per, m_per), :]
            return jnp.dot(xs, w_ref[:, :], preferred_element_type=jnp.float32)

        comm_ref[0] = partial_chunk(left)

        for s in range(N_DEV - 1):
            send_slot = s % 2
            recv_slot = (s + 1) % 2
            rdma = pltpu.make_async_remote_copy(
                src_ref=comm_ref.at[send_slot],
                dst_ref=comm_ref.at[recv_slot],
                send_sem=send_sems.at[send_slot],
                recv_sem=recv_sems.at[recv_slot],
                device_id=(right,),
                device_id_type=pl.DeviceIdType.MESH,
            )
            rdma.start()
            rdma.wait()
            c_recv = lax.rem(my + 2 * N_DEV - s - 2, N_DEV)
            comm_ref[recv_slot] = comm_ref[recv_slot] + partial_chunk(c_recv)

        y = comm_ref[(N_DEV - 1) % 2]
        out_ref[:, :] = y * jax.nn.sigmoid(y)

    return pl.pallas_call(
        body,
        out_shape=jax.ShapeDtypeStruct((m_per, n), jnp.float32),
        in_specs=[
            pl.BlockSpec(memory_space=pltpu.VMEM),
            pl.BlockSpec(memory_space=pltpu.VMEM),
        ],
        out_specs=pl.BlockSpec(memory_space=pltpu.VMEM),
        scratch_shapes=[
            pltpu.VMEM((2, m_per, n), jnp.float32),
            pltpu.SemaphoreType.DMA((2,)),
            pltpu.SemaphoreType.DMA((2,)),
        ],
        compiler_params=pltpu.CompilerParams(collective_id=0),
    )(x, w_mat)


# device time: 422841 ns/iter; 3.3434x vs baseline; 3.3434x over previous
import jax
import jax.numpy as jnp
from jax import lax
from jax.experimental import pallas as pl
from jax.experimental.pallas import tpu as pltpu

N_DEV = 16


def kernel(x, w_mat):
    m, k_per = x.shape
    _, n = w_mat.shape
    m_per = m // N_DEV
    nh = n // 2

    def body(x_ref, w_ref, out_ref, comm_r, comm_l,
             send_r, recv_r, send_l, recv_l):
        my = lax.axis_index("i")
        right = lax.rem(my + 1, N_DEV)
        left = lax.rem(my + N_DEV - 1, N_DEV)

        barrier_sem = pltpu.get_barrier_semaphore()
        for nbr in (left, right):
            pl.semaphore_signal(
                barrier_sem, inc=1,
                device_id=(nbr,), device_id_type=pl.DeviceIdType.MESH,
            )
        pl.semaphore_wait(barrier_sem, 2)

        def partial_r(c):
            xs = x_ref[pl.ds(c * m_per, m_per), :]
            return jnp.dot(xs, w_ref[:, :nh], preferred_element_type=jnp.float32)

        def partial_l(c):
            xs = x_ref[pl.ds(c * m_per, m_per), :]
            return jnp.dot(xs, w_ref[:, nh:], preferred_element_type=jnp.float32)

        comm_r[0] = partial_r(left).astype(jnp.bfloat16)
        comm_l[0] = partial_l(right).astype(jnp.bfloat16)

        for s in range(N_DEV - 1):
            send_slot = s % 2
            recv_slot = (s + 1) % 2
            rdma_r = pltpu.make_async_remote_copy(
                src_ref=comm_r.at[send_slot],
                dst_ref=comm_r.at[recv_slot],
                send_sem=send_r.at[send_slot],
                recv_sem=recv_r.at[recv_slot],
                device_id=(right,),
                device_id_type=pl.DeviceIdType.MESH,
            )
            rdma_l = pltpu.make_async_remote_copy(
                src_ref=comm_l.at[send_slot],
                dst_ref=comm_l.at[recv_slot],
                send_sem=send_l.at[send_slot],
                recv_sem=recv_l.at[recv_slot],
                device_id=(left,),
                device_id_type=pl.DeviceIdType.MESH,
            )
            rdma_r.start()
            rdma_l.start()

            c_r = lax.rem(my + 2 * N_DEV - s - 2, N_DEV)
            c_l = lax.rem(my + s + 2, N_DEV)
            p_r = partial_r(c_r)
            p_l = partial_l(c_l)

            rdma_r.wait()
            rdma_l.wait()

            if s < N_DEV - 2:
                comm_r[recv_slot] = (
                    comm_r[recv_slot].astype(jnp.float32) + p_r
                ).astype(jnp.bfloat16)
                comm_l[recv_slot] = (
                    comm_l[recv_slot].astype(jnp.float32) + p_l
                ).astype(jnp.bfloat16)
            else:
                y_r = comm_r[recv_slot].astype(jnp.float32) + p_r
                y_l = comm_l[recv_slot].astype(jnp.float32) + p_l
                out_ref[:, :nh] = y_r * jax.nn.sigmoid(y_r)
                out_ref[:, nh:] = y_l * jax.nn.sigmoid(y_l)

    return pl.pallas_call(
        body,
        out_shape=jax.ShapeDtypeStruct((m_per, n), jnp.float32),
        in_specs=[
            pl.BlockSpec(memory_space=pltpu.VMEM),
            pl.BlockSpec(memory_space=pltpu.VMEM),
        ],
        out_specs=pl.BlockSpec(memory_space=pltpu.VMEM),
        scratch_shapes=[
            pltpu.VMEM((2, m_per, nh), jnp.bfloat16),
            pltpu.VMEM((2, m_per, nh), jnp.bfloat16),
            pltpu.SemaphoreType.DMA((2,)),
            pltpu.SemaphoreType.DMA((2,)),
            pltpu.SemaphoreType.DMA((2,)),
            pltpu.SemaphoreType.DMA((2,)),
        ],
        compiler_params=pltpu.CompilerParams(collective_id=0),
    )(x, w_mat)


# device time: 356169 ns/iter; 3.9692x vs baseline; 1.1872x over previous
import jax
import jax.numpy as jnp
from jax import lax
from jax.experimental import pallas as pl
from jax.experimental.pallas import tpu as pltpu

N_DEV = 16
N_STEP = N_DEV - 1


def kernel(x, w_mat):
    m, k_per = x.shape
    _, n = w_mat.shape
    m_per = m // N_DEV
    nh = n // 2
    nq = n // 4

    def body(x_ref, w_ref, out_ref,
             comm_r0, comm_r1, comm_l0, comm_l1,
             send_r0, recv_r0, send_r1, recv_r1,
             send_l0, recv_l0, send_l1, recv_l1):
        my = lax.axis_index("i")
        right = lax.rem(my + 1, N_DEV)
        left = lax.rem(my + N_DEV - 1, N_DEV)

        def partial(c, col0, col1):
            xs = x_ref[pl.ds(c * m_per, m_per), :]
            return jnp.dot(
                xs, w_ref[:, col0:col1], preferred_element_type=jnp.float32
            )

        rings = {
            "r0": (comm_r0, send_r0, recv_r0, right, 0),
            "r1": (comm_r1, send_r1, recv_r1, right, nq),
            "l0": (comm_l0, send_l0, recv_l0, left, nh),
            "l1": (comm_l1, send_l1, recv_l1, left, nh + nq),
        }

        def desc(name, t):
            buf, ssem, rsem, peer, _ = rings[name]
            return pltpu.make_async_remote_copy(
                src_ref=buf.at[t % 2],
                dst_ref=buf.at[(t + 1) % 2],
                send_sem=ssem.at[t % 2],
                recv_sem=rsem.at[(t + 1) % 2],
                device_id=(peer,),
                device_id_type=pl.DeviceIdType.MESH,
            )

        p_r = partial(left, 0, nh)
        comm_r0[0] = p_r[:, :nq].astype(jnp.bfloat16)
        comm_r1[0] = p_r[:, nq:].astype(jnp.bfloat16)
        p_l = partial(right, nh, n)
        comm_l0[0] = p_l[:, :nq].astype(jnp.bfloat16)
        comm_l1[0] = p_l[:, nq:].astype(jnp.bfloat16)

        barrier_sem = pltpu.get_barrier_semaphore()
        for nbr in (left, right):
            pl.semaphore_signal(
                barrier_sem, inc=1,
                device_id=(nbr,), device_id_type=pl.DeviceIdType.MESH,
            )
        pl.semaphore_wait(barrier_sem, 2)

        descs = {name: {0: desc(name, 0)} for name in rings}
        for name in rings:
            descs[name][0].start()

        for t in range(N_STEP):
            slot = (t + 1) % 2
            c_r = lax.rem(my + 2 * N_DEV - t - 2, N_DEV)
            c_l = lax.rem(my + t + 2, N_DEV)
            p_r = partial(c_r, 0, nh)
            p_l = partial(c_l, nh, n)
            quarters = (
                ("r0", p_r[:, :nq]), ("l0", p_l[:, :nq]),
                ("r1", p_r[:, nq:]), ("l1", p_l[:, nq:]),
            )
            if t < N_STEP - 1:
                for name, p in quarters:
                    buf = rings[name][0]
                    descs[name][t].wait_recv()
                    if t > 0:
                        descs[name][t - 1].wait_send()
                    buf[slot] = (buf[slot].astype(jnp.float32) + p).astype(
                        jnp.bfloat16
                    )
                    d = desc(name, t + 1)
                    descs[name][t + 1] = d
                    d.start()
            else:
                for name, p in quarters:
                    buf, _, _, _, col = rings[name]
                    descs[name][t].wait_recv()
                    y = buf[slot].astype(jnp.float32) + p
                    out_ref[:, col : col + nq] = y * jax.nn.sigmoid(y)
                for name in rings:
                    descs[name][t - 1].wait_send()
                    descs[name][t].wait_send()

    return pl.pallas_call(
        body,
        out_shape=jax.ShapeDtypeStruct((m_per, n), jnp.float32),
        in_specs=[
            pl.BlockSpec(memory_space=pltpu.VMEM),
            pl.BlockSpec(memory_space=pltpu.VMEM),
        ],
        out_specs=pl.BlockSpec(memory_space=pltpu.VMEM),
        scratch_shapes=[
            pltpu.VMEM((2, m_per, nq), jnp.bfloat16),
            pltpu.VMEM((2, m_per, nq), jnp.bfloat16),
            pltpu.VMEM((2, m_per, nq), jnp.bfloat16),
            pltpu.VMEM((2, m_per, nq), jnp.bfloat16),
            pltpu.SemaphoreType.DMA((2,)),
            pltpu.SemaphoreType.DMA((2,)),
            pltpu.SemaphoreType.DMA((2,)),
            pltpu.SemaphoreType.DMA((2,)),
            pltpu.SemaphoreType.DMA((2,)),
            pltpu.SemaphoreType.DMA((2,)),
            pltpu.SemaphoreType.DMA((2,)),
            pltpu.SemaphoreType.DMA((2,)),
        ],
        compiler_params=pltpu.CompilerParams(collective_id=0),
    )(x, w_mat)


# device time: 355832 ns/iter; 3.9730x vs baseline; 1.0009x over previous
import jax
import jax.numpy as jnp
from jax import lax
from jax.experimental import pallas as pl
from jax.experimental.pallas import tpu as pltpu

N_DEV = 16
N_STEP = N_DEV - 1
S = 4


def kernel(x, w_mat):
    m, k_per = x.shape
    _, n = w_mat.shape
    m_per = m // N_DEV
    nh = n // 2
    nsc = nh // S

    ring_names = [f"{d}{j}" for j in range(S) for d in ("r", "l")]

    def body(x_ref, w_ref, out_ref, *scratch):
        bufs = scratch[: 2 * S]
        sems = scratch[2 * S :]
        my = lax.axis_index("i")
        right = lax.rem(my + 1, N_DEV)
        left = lax.rem(my + N_DEV - 1, N_DEV)

        rings = {}
        for idx, name in enumerate(ring_names):
            d, j = name[0], int(name[1])
            col = (0 if d == "r" else nh) + j * nsc
            peer = right if d == "r" else left
            rings[name] = (bufs[idx], sems[2 * idx], sems[2 * idx + 1], peer, col)

        def partial(c, col0, col1):
            xs = x_ref[pl.ds(c * m_per, m_per), :]
            return jnp.dot(
                xs, w_ref[:, col0:col1], preferred_element_type=jnp.float32
            )

        def desc(name, t):
            buf, ssem, rsem, peer, _ = rings[name]
            return pltpu.make_async_remote_copy(
                src_ref=buf.at[t % 2],
                dst_ref=buf.at[(t + 1) % 2],
                send_sem=ssem.at[t % 2],
                recv_sem=rsem.at[(t + 1) % 2],
                device_id=(peer,),
                device_id_type=pl.DeviceIdType.MESH,
            )

        p_r = partial(left, 0, nh)
        p_l = partial(right, nh, n)
        for name in ring_names:
            buf, _, _, _, col = rings[name]
            p = p_r if name[0] == "r" else p_l
            c0 = col if name[0] == "r" else col - nh
            buf[0] = p[:, c0 : c0 + nsc].astype(jnp.bfloat16)

        barrier_sem = pltpu.get_barrier_semaphore()
        for nbr in (left, right):
            pl.semaphore_signal(
                barrier_sem, inc=1,
                device_id=(nbr,), device_id_type=pl.DeviceIdType.MESH,
            )
        pl.semaphore_wait(barrier_sem, 2)

        descs = {name: {0: desc(name, 0)} for name in ring_names}
        for name in ring_names:
            descs[name][0].start()

        for t in range(N_STEP):
            slot = (t + 1) % 2
            c_r = lax.rem(my + 2 * N_DEV - t - 2, N_DEV)
            c_l = lax.rem(my + t + 2, N_DEV)
            p_r = partial(c_r, 0, nh)
            p_l = partial(c_l, nh, n)
            if t < N_STEP - 1:
                for name in ring_names:
                    buf, _, _, _, col = rings[name]
                    p = p_r if name[0] == "r" else p_l
                    c0 = col if name[0] == "r" else col - nh
                    descs[name][t].wait_recv()
                    if t > 0:
                        descs[name][t - 1].wait_send()
                    buf[slot] = (
                        buf[slot].astype(jnp.float32) + p[:, c0 : c0 + nsc]
                    ).astype(jnp.bfloat16)
                    d = desc(name, t + 1)
                    descs[name][t + 1] = d
                    d.start()
            else:
                for name in ring_names:
                    buf, _, _, _, col = rings[name]
                    p = p_r if name[0] == "r" else p_l
                    c0 = col if name[0] == "r" else col - nh
                    descs[name][t].wait_recv()
                    y = buf[slot].astype(jnp.float32) + p[:, c0 : c0 + nsc]
                    out_ref[:, col : col + nsc] = y * jax.nn.sigmoid(y)
                for name in ring_names:
                    descs[name][t - 1].wait_send()
                    descs[name][t].wait_send()

    scratch_shapes = [
        pltpu.VMEM((2, m_per, nsc), jnp.bfloat16) for _ in range(2 * S)
    ]
    for _ in range(2 * S):
        scratch_shapes.append(pltpu.SemaphoreType.DMA((2,)))
        scratch_shapes.append(pltpu.SemaphoreType.DMA((2,)))

    return pl.pallas_call(
        body,
        out_shape=jax.ShapeDtypeStruct((m_per, n), jnp.float32),
        in_specs=[
            pl.BlockSpec(memory_space=pltpu.VMEM),
            pl.BlockSpec(memory_space=pltpu.VMEM),
        ],
        out_specs=pl.BlockSpec(memory_space=pltpu.VMEM),
        scratch_shapes=scratch_shapes,
        compiler_params=pltpu.CompilerParams(collective_id=0),
    )(x, w_mat)


# device time: 352870 ns/iter; 4.0063x vs baseline; 1.0084x over previous
import jax
import jax.numpy as jnp
from jax import lax
from jax.experimental import pallas as pl
from jax.experimental.pallas import tpu as pltpu

N_DEV = 16
N_STEP = N_DEV - 1
S = 4


def kernel(x, w_mat):
    m, k_per = x.shape
    _, n = w_mat.shape
    m_per = m // N_DEV
    nh = n // 2
    nsc = nh // S

    ring_names = [f"{d}{j}" for j in range(S) for d in ("r", "l")]

    def body(x_ref, w_ref, out_ref, *scratch):
        bufs = scratch[: 2 * S]
        sems = scratch[2 * S :]
        my = lax.axis_index("i")

        def sigma(rr):
            col = rr // 4
            zp = lax.rem(rr, 4)
            zz = jnp.where(lax.rem(col, 2) == 0, zp, 3 - zp)
            return 4 * zz + col

        colp = lax.rem(my, 4)
        zpos = jnp.where(lax.rem(colp, 2) == 0, my // 4, 3 - my // 4)
        rpos = 4 * colp + zpos

        right = sigma(lax.rem(rpos + 1, N_DEV))
        left = sigma(lax.rem(rpos + N_DEV - 1, N_DEV))

        rings = {}
        for idx, name in enumerate(ring_names):
            d, j = name[0], int(name[1])
            col = (0 if d == "r" else nh) + j * nsc
            peer = right if d == "r" else left
            rings[name] = (bufs[idx], sems[2 * idx], sems[2 * idx + 1], peer, col)

        def partial(c, col0, col1):
            xs = x_ref[pl.ds(c * m_per, m_per), :]
            return jnp.dot(
                xs, w_ref[:, col0:col1], preferred_element_type=jnp.float32
            )

        def desc(name, t):
            buf, ssem, rsem, peer, _ = rings[name]
            return pltpu.make_async_remote_copy(
                src_ref=buf.at[t % 2],
                dst_ref=buf.at[(t + 1) % 2],
                send_sem=ssem.at[t % 2],
                recv_sem=rsem.at[(t + 1) % 2],
                device_id=(peer,),
                device_id_type=pl.DeviceIdType.MESH,
            )

        p_r = partial(left, 0, nh)
        p_l = partial(right, nh, n)
        for name in ring_names:
            buf, _, _, _, col = rings[name]
            p = p_r if name[0] == "r" else p_l
            c0 = col if name[0] == "r" else col - nh
            buf[0] = p[:, c0 : c0 + nsc].astype(jnp.bfloat16)

        barrier_sem = pltpu.get_barrier_semaphore()
        for nbr in (left, right):
            pl.semaphore_signal(
                barrier_sem, inc=1,
                device_id=(nbr,), device_id_type=pl.DeviceIdType.MESH,
            )
        pl.semaphore_wait(barrier_sem, 2)

        descs = {name: {0: desc(name, 0)} for name in ring_names}
        for name in ring_names:
            descs[name][0].start()

        for t in range(N_STEP):
            slot = (t + 1) % 2
            c_r = sigma(lax.rem(rpos + 2 * N_DEV - t - 2, N_DEV))
            c_l = sigma(lax.rem(rpos + t + 2, N_DEV))
            p_r = partial(c_r, 0, nh)
            p_l = partial(c_l, nh, n)
            if t < N_STEP - 1:
                for name in ring_names:
                    buf, _, _, _, col = rings[name]
                    p = p_r if name[0] == "r" else p_l
                    c0 = col if name[0] == "r" else col - nh
                    descs[name][t].wait_recv()
                    if t > 0:
                        descs[name][t - 1].wait_send()
                    buf[slot] = (
                        buf[slot].astype(jnp.float32) + p[:, c0 : c0 + nsc]
                    ).astype(jnp.bfloat16)
                    d = desc(name, t + 1)
                    descs[name][t + 1] = d
                    d.start()
            else:
                for name in ring_names:
                    buf, _, _, _, col = rings[name]
                    p = p_r if name[0] == "r" else p_l
                    c0 = col if name[0] == "r" else col - nh
                    descs[name][t].wait_recv()
                    y = buf[slot].astype(jnp.float32) + p[:, c0 : c0 + nsc]
                    out_ref[:, col : col + nsc] = y * jax.nn.sigmoid(y)
                for name in ring_names:
                    descs[name][t - 1].wait_send()
                    descs[name][t].wait_send()

    scratch_shapes = [
        pltpu.VMEM((2, m_per, nsc), jnp.bfloat16) for _ in range(2 * S)
    ]
    for _ in range(2 * S):
        scratch_shapes.append(pltpu.SemaphoreType.DMA((2,)))
        scratch_shapes.append(pltpu.SemaphoreType.DMA((2,)))

    return pl.pallas_call(
        body,
        out_shape=jax.ShapeDtypeStruct((m_per, n), jnp.float32),
        in_specs=[
            pl.BlockSpec(memory_space=pltpu.VMEM),
            pl.BlockSpec(memory_space=pltpu.VMEM),
        ],
        out_specs=pl.BlockSpec(memory_space=pltpu.VMEM),
        scratch_shapes=scratch_shapes,
        compiler_params=pltpu.CompilerParams(collective_id=0),
    )(x, w_mat)


# device time: 352620 ns/iter; 4.0092x vs baseline; 1.0007x over previous
import jax
import jax.numpy as jnp
from jax import lax
from jax.experimental import pallas as pl
from jax.experimental.pallas import tpu as pltpu

N_DEV = 16
N_STEP = N_DEV - 1
S = 4


def kernel(x, w_mat):
    m, k_per = x.shape
    _, n = w_mat.shape
    m_per = m // N_DEV
    nh = n // 2
    nsc = nh // S

    ring_names = [f"{d}{j}" for j in range(S) for d in ("r", "l")]

    def body(x_ref, w_ref, out_ref, *scratch):
        bufs = scratch[: 2 * S]
        sems = scratch[2 * S :]
        my = lax.axis_index("i")

        def sigma(rr):
            col = rr // 4
            zp = lax.rem(rr, 4)
            zz = jnp.where(lax.rem(col, 2) == 0, zp, 3 - zp)
            return 4 * zz + col

        colp = lax.rem(my, 4)
        zpos = jnp.where(lax.rem(colp, 2) == 0, my // 4, 3 - my // 4)
        rpos = 4 * colp + zpos

        right = sigma(lax.rem(rpos + 1, N_DEV))
        left = sigma(lax.rem(rpos + N_DEV - 1, N_DEV))

        rings = {}
        for idx, name in enumerate(ring_names):
            d, j = name[0], int(name[1])
            col = (0 if d == "r" else nh) + j * nsc
            peer = right if d == "r" else left
            rings[name] = (bufs[idx], sems[2 * idx], sems[2 * idx + 1], peer, col)

        def partial(c, col0, col1):
            xs = x_ref[pl.ds(c * m_per, m_per), :]
            return jnp.dot(
                xs, w_ref[:, col0:col1], preferred_element_type=jnp.float32
            )

        def desc(name, t):
            buf, ssem, rsem, peer, _ = rings[name]
            return pltpu.make_async_remote_copy(
                src_ref=buf.at[t % 2],
                dst_ref=buf.at[(t + 1) % 2],
                send_sem=ssem.at[t % 2],
                recv_sem=rsem.at[(t + 1) % 2],
                device_id=(peer,),
                device_id_type=pl.DeviceIdType.MESH,
            )

        barrier_sem = pltpu.get_barrier_semaphore()
        for nbr in (left, right):
            pl.semaphore_signal(
                barrier_sem, inc=1,
                device_id=(nbr,), device_id_type=pl.DeviceIdType.MESH,
            )
        pl.semaphore_wait(barrier_sem, 2)

        descs = {}
        p_r = partial(left, 0, nh)
        for j in range(S):
            name = f"r{j}"
            buf = rings[name][0]
            buf[0] = p_r[:, j * nsc : (j + 1) * nsc].astype(jnp.bfloat16)
            descs[name] = {0: desc(name, 0)}
            descs[name][0].start()
        p_l = partial(right, nh, n)
        for j in range(S):
            name = f"l{j}"
            buf = rings[name][0]
            buf[0] = p_l[:, j * nsc : (j + 1) * nsc].astype(jnp.bfloat16)
            descs[name] = {0: desc(name, 0)}
            descs[name][0].start()

        for t in range(N_STEP):
            slot = (t + 1) % 2
            c_r = sigma(lax.rem(rpos + 2 * N_DEV - t - 2, N_DEV))
            c_l = sigma(lax.rem(rpos + t + 2, N_DEV))
            p_r = partial(c_r, 0, nh)
            p_l = partial(c_l, nh, n)
            if t < N_STEP - 1:
                for name in ring_names:
                    buf, _, _, _, col = rings[name]
                    p = p_r if name[0] == "r" else p_l
                    c0 = col if name[0] == "r" else col - nh
                    descs[name][t].wait_recv()
                    if t > 0:
                        descs[name][t - 1].wait_send()
                    buf[slot] = (
                        buf[slot].astype(jnp.float32) + p[:, c0 : c0 + nsc]
                    ).astype(jnp.bfloat16)
                    d = desc(name, t + 1)
                    descs[name][t + 1] = d
                    d.start()
            else:
                for name in ring_names:
                    buf, _, _, _, col = rings[name]
                    p = p_r if name[0] == "r" else p_l
                    c0 = col if name[0] == "r" else col - nh
                    descs[name][t].wait_recv()
                    y = buf[slot].astype(jnp.float32) + p[:, c0 : c0 + nsc]
                    out_ref[:, col : col + nsc] = y * jax.nn.sigmoid(y)
                for name in ring_names:
                    descs[name][t - 1].wait_send()
                    descs[name][t].wait_send()

    scratch_shapes = [
        pltpu.VMEM((2, m_per, nsc), jnp.bfloat16) for _ in range(2 * S)
    ]
    for _ in range(2 * S):
        scratch_shapes.append(pltpu.SemaphoreType.DMA((2,)))
        scratch_shapes.append(pltpu.SemaphoreType.DMA((2,)))

    return pl.pallas_call(
        body,
        out_shape=jax.ShapeDtypeStruct((m_per, n), jnp.float32),
        in_specs=[
            pl.BlockSpec(memory_space=pltpu.VMEM),
            pl.BlockSpec(memory_space=pltpu.VMEM),
        ],
        out_specs=pl.BlockSpec(memory_space=pltpu.VMEM),
        scratch_shapes=scratch_shapes,
        compiler_params=pltpu.CompilerParams(collective_id=0),
    )(x, w_mat)
